# baseline (speedup 1.0000x reference)
import jax
import jax.numpy as jnp
from jax import lax
from jax.experimental import pallas as pl
from jax.experimental.pallas import tpu as pltpu
from jax.experimental.pallas import tpu_sc as plsc

EMBED = 64
LANES = 16
NC, NS = 2, 16
NW = NC * NS

ROWS = 16384 * 200
CHUNK = 640
NCHUNK = ROWS // (NW * CHUNK)
GATHER = 128
NGATHER = CHUNK // GATHER


def _body(x_hbm, table_hbm, out_hbm, xbuf, pidx, rows, sem_in, sem_g, sem_out):
    wid = lax.axis_index("s") * NC + lax.axis_index("c")
    base = wid * CHUNK

    def rowbase(ci):
        return base + ci * (NW * CHUNK)

    def in_copy(ci):
        p = lax.rem(ci, 2)
        return pltpu.make_async_copy(
            x_hbm.at[pl.ds(rowbase(ci), CHUNK)],
            xbuf.at[p],
            sem_in.at[p],
        )

    def gather_copy(ci, j):
        p = lax.rem(ci, 2)
        return pltpu.make_async_copy(
            table_hbm.at[pidx.at[p, j]],
            rows.at[p, pl.ds(j * GATHER, GATHER)],
            sem_g.at[p],
        )

    def out_copy(ci):
        p = lax.rem(ci, 2)
        return pltpu.make_async_copy(
            rows.at[p],
            out_hbm.at[pl.ds(rowbase(ci), CHUNK)],
            sem_out.at[p],
        )

    in_copy(0).start()

    def chunk_body(ci, carry):
        p = lax.rem(ci, 2)

        @pl.when(ci < NCHUNK - 1)
        def _prefetch():
            in_copy(ci + 1).start()

        @pl.when(ci >= 1)
        def _drain_prev():
            for j in range(NGATHER):
                gather_copy(ci - 1, j).wait()
            out_copy(ci - 1).start()

        in_copy(ci).wait()
        for t in range(CHUNK // LANES):
            j, k = divmod(t, GATHER // LANES)
            v = xbuf[p, pl.ds(LANES * t, LANES)]
            pidx[p, j, pl.ds(k * LANES, LANES)] = lax.rem(v, 7)

        @pl.when(ci >= 2)
        def _free_rows():
            out_copy(ci - 2).wait()

        for j in range(NGATHER):
            gather_copy(ci, j).start()
        return carry

    lax.fori_loop(0, NCHUNK, chunk_body, 0)

    for j in range(NGATHER):
        gather_copy(NCHUNK - 1, j).wait()
    out_copy(NCHUNK - 1).start()
    out_copy(NCHUNK - 2).wait()
    out_copy(NCHUNK - 1).wait()


def kernel(x, table):
    x_flat = x.reshape(ROWS).astype(jnp.int32)
    mesh = plsc.VectorSubcoreMesh(core_axis_name="c", subcore_axis_name="s")
    out = pl.kernel(
        _body,
        out_type=jax.ShapeDtypeStruct((ROWS, EMBED), jnp.float32),
        mesh=mesh,
        compiler_params=pltpu.CompilerParams(use_tc_tiling_on_sc=False),
        scratch_types=[
            pltpu.VMEM((2, CHUNK), jnp.int32),
            pltpu.VMEM((2, NGATHER, GATHER), jnp.int32),
            pltpu.VMEM((2, CHUNK, EMBED), jnp.float32),
            pltpu.SemaphoreType.DMA((2,)),
            pltpu.SemaphoreType.DMA((2,)),
            pltpu.SemaphoreType.DMA((2,)),
        ],
    )(x_flat, table)
    return out.reshape(x.shape[0], x.shape[1], EMBED)


# speedup vs baseline: 1.0012x; 1.0012x over previous
import jax
import jax.numpy as jnp
from jax import lax
from jax.experimental import pallas as pl
from jax.experimental.pallas import tpu as pltpu
from jax.experimental.pallas import tpu_sc as plsc

EMBED = 64
LANES = 16
NC, NS = 2, 16
NW = NC * NS

ROWS = 16384 * 200
CHUNK = 640
NCHUNK = ROWS // (NW * CHUNK)
GATHER = 128
NGATHER = CHUNK // GATHER


def _body(x_hbm, table_hbm, out_hbm, xbuf, pidx, rows, sem_in, sem_g, sem_out):
    wid = lax.axis_index("s") * NC + lax.axis_index("c")
    base = wid * CHUNK

    def rowbase(ci):
        return base + ci * (NW * CHUNK)

    def in_copy(ci):
        p = lax.rem(ci, 2)
        return pltpu.make_async_copy(
            x_hbm.at[pl.ds(rowbase(ci), CHUNK)],
            xbuf.at[p],
            sem_in.at[p],
        )

    def gather_copy(ci, j):
        del j
        p = lax.rem(ci, 2)
        return pltpu.make_async_copy(
            table_hbm.at[pidx.at[p]],
            rows.at[p],
            sem_g.at[p],
        )

    def out_copy(ci):
        p = lax.rem(ci, 2)
        return pltpu.make_async_copy(
            rows.at[p],
            out_hbm.at[pl.ds(rowbase(ci), CHUNK)],
            sem_out.at[p],
        )

    in_copy(0).start()

    def chunk_body(ci, carry):
        p = lax.rem(ci, 2)

        @pl.when(ci < NCHUNK - 1)
        def _prefetch():
            in_copy(ci + 1).start()

        @pl.when(ci >= 1)
        def _drain_prev():
            gather_copy(ci - 1, 0).wait()
            out_copy(ci - 1).start()

        in_copy(ci).wait()
        for t in range(CHUNK // LANES):
            v = xbuf[p, pl.ds(LANES * t, LANES)]
            pidx[p, pl.ds(LANES * t, LANES)] = lax.rem(v, 7)

        @pl.when(ci >= 2)
        def _free_rows():
            out_copy(ci - 2).wait()

        gather_copy(ci, 0).start()
        return carry

    lax.fori_loop(0, NCHUNK, chunk_body, 0)

    gather_copy(NCHUNK - 1, 0).wait()
    out_copy(NCHUNK - 1).start()
    out_copy(NCHUNK - 2).wait()
    out_copy(NCHUNK - 1).wait()


def kernel(x, table):
    x_flat = x.reshape(ROWS).astype(jnp.int32)
    mesh = plsc.VectorSubcoreMesh(core_axis_name="c", subcore_axis_name="s")
    out = pl.kernel(
        _body,
        out_type=jax.ShapeDtypeStruct((ROWS, EMBED), jnp.float32),
        mesh=mesh,
        compiler_params=pltpu.CompilerParams(use_tc_tiling_on_sc=False),
        scratch_types=[
            pltpu.VMEM((2, CHUNK), jnp.int32),
            pltpu.VMEM((2, CHUNK), jnp.int32),
            pltpu.VMEM((2, CHUNK, EMBED), jnp.float32),
            pltpu.SemaphoreType.DMA((2,)),
            pltpu.SemaphoreType.DMA((2,)),
            pltpu.SemaphoreType.DMA((2,)),
        ],
    )(x_flat, table)
    return out.reshape(x.shape[0], x.shape[1], EMBED)


# speedup vs baseline: 5.7820x; 5.7751x over previous
import jax
import jax.numpy as jnp
from jax import lax
from jax.experimental import pallas as pl
from jax.experimental.pallas import tpu as pltpu
from jax.experimental.pallas import tpu_sc as plsc

EMBED = 64
LANES = 16
NC, NS = 2, 16
NW = NC * NS

ROWS = 16384 * 200
CHUNK = 640
NCHUNK = ROWS // (NW * CHUNK)
UNROLL = 4


def _body(x_hbm, table_hbm, out_hbm, tv, xbuf, rows, sem_in, sem_out):
    wid = lax.axis_index("s") * NC + lax.axis_index("c")
    base = wid * CHUNK

    def rowbase(ci):
        return base + ci * (NW * CHUNK)

    def in_copy(ci):
        p = lax.rem(ci, 2)
        return pltpu.make_async_copy(
            x_hbm.at[pl.ds(rowbase(ci), CHUNK)],
            xbuf.at[p],
            sem_in.at[p],
        )

    def out_copy(ci):
        p = lax.rem(ci, 2)
        return pltpu.make_async_copy(
            rows.at[p],
            out_hbm.at[pl.ds(rowbase(ci), CHUNK)],
            sem_out.at[p],
        )

    pltpu.sync_copy(table_hbm, tv)
    in_copy(0).start()

    def chunk_body(ci, carry):
        p = lax.rem(ci, 2)

        @pl.when(ci < NCHUNK - 1)
        def _prefetch():
            in_copy(ci + 1).start()

        @pl.when(ci >= 2)
        def _free_rows():
            out_copy(ci - 2).wait()

        in_copy(ci).wait()

        def fill(b, carry):
            r = lax.rem(xbuf[p, pl.ds(b * LANES, LANES)], 7)
            for u in range(LANES):
                i = b * LANES + u
                ri = r[u]
                for g in range(EMBED // LANES):
                    rows[p, i, pl.ds(g * LANES, LANES)] = (
                        tv[ri, pl.ds(g * LANES, LANES)]
                    )
            return carry

        lax.fori_loop(0, CHUNK // LANES, fill, 0)
        out_copy(ci).start()
        return carry

    lax.fori_loop(0, NCHUNK, chunk_body, 0)

    out_copy(NCHUNK - 2).wait()
    out_copy(NCHUNK - 1).wait()


def kernel(x, table):
    x_flat = x.reshape(ROWS).astype(jnp.int32)
    mesh = plsc.VectorSubcoreMesh(core_axis_name="c", subcore_axis_name="s")
    out = pl.kernel(
        _body,
        out_type=jax.ShapeDtypeStruct((ROWS, EMBED), jnp.float32),
        mesh=mesh,
        compiler_params=pltpu.CompilerParams(use_tc_tiling_on_sc=False),
        scratch_types=[
            pltpu.VMEM((7, EMBED), jnp.float32),
            pltpu.VMEM((2, CHUNK), jnp.int32),
            pltpu.VMEM((2, CHUNK, EMBED), jnp.float32),
            pltpu.SemaphoreType.DMA((2,)),
            pltpu.SemaphoreType.DMA((2,)),
        ],
    )(x_flat, table)
    return out.reshape(x.shape[0], x.shape[1], EMBED)


# speedup vs baseline: 8.9607x; 1.5498x over previous
import jax
import jax.numpy as jnp
from jax import lax
from jax.experimental import pallas as pl
from jax.experimental.pallas import tpu as pltpu
from jax.experimental.pallas import tpu_sc as plsc

EMBED = 64
LANES = 16
NC, NS = 2, 16
NW = NC * NS

ROWS = 16384 * 200
CHUNK = 640
NCHUNK = ROWS // (NW * CHUNK)
UNROLL = 4


def _body(x_hbm, table_hbm, out_hbm, tv, xbuf, rows, sem_in, sem_out):
    wid = lax.axis_index("s") * NC + lax.axis_index("c")
    base = wid * CHUNK

    def rowbase(ci):
        return base + ci * (NW * CHUNK)

    def in_copy(ci):
        p = lax.rem(ci, 2)
        return pltpu.make_async_copy(
            x_hbm.at[pl.ds(rowbase(ci), CHUNK)],
            xbuf.at[p],
            sem_in.at[p],
        )

    def out_copy(ci):
        p = lax.rem(ci, 2)
        return pltpu.make_async_copy(
            rows.at[p],
            out_hbm.at[pl.ds(rowbase(ci), CHUNK)],
            sem_out.at[p],
        )

    pltpu.sync_copy(table_hbm, tv)
    in_copy(0).start()

    def chunk_body(ci, carry):
        p = lax.rem(ci, 2)

        @pl.when(ci < NCHUNK - 1)
        def _prefetch():
            in_copy(ci + 1).start()

        @pl.when(ci >= 2)
        def _free_rows():
            out_copy(ci - 2).wait()

        in_copy(ci).wait()

        @plsc.parallel_loop(0, CHUNK // LANES, unroll=2)
        def _fill(b):
            r = lax.rem(xbuf[p, pl.ds(b * LANES, LANES)], 7)
            for u in range(LANES):
                i = b * LANES + u
                ri = r[u]
                for g in range(EMBED // LANES):
                    rows[p, i, pl.ds(g * LANES, LANES)] = (
                        tv[ri, pl.ds(g * LANES, LANES)]
                    )
        out_copy(ci).start()
        return carry

    lax.fori_loop(0, NCHUNK, chunk_body, 0)

    out_copy(NCHUNK - 2).wait()
    out_copy(NCHUNK - 1).wait()


def kernel(x, table):
    x_flat = x.reshape(ROWS).astype(jnp.int32)
    mesh = plsc.VectorSubcoreMesh(core_axis_name="c", subcore_axis_name="s")
    out = pl.kernel(
        _body,
        out_type=jax.ShapeDtypeStruct((ROWS, EMBED), jnp.float32),
        mesh=mesh,
        compiler_params=pltpu.CompilerParams(use_tc_tiling_on_sc=False),
        scratch_types=[
            pltpu.VMEM((7, EMBED), jnp.float32),
            pltpu.VMEM((2, CHUNK), jnp.int32),
            pltpu.VMEM((2, CHUNK, EMBED), jnp.float32),
            pltpu.SemaphoreType.DMA((2,)),
            pltpu.SemaphoreType.DMA((2,)),
        ],
    )(x_flat, table)
    return out.reshape(x.shape[0], x.shape[1], EMBED)
